# baseline (speedup 1.0000x reference)
import functools

import jax
import jax.numpy as jnp
from jax import lax
from jax.experimental import pallas as pl
from jax.experimental.pallas import tpu as pltpu
from jax.experimental.pallas import tpu_sc as plsc

N = 10000
NP = 10240
E = 320000
F = 128
NT = 16
CH = 128
BI = 16
NSB = 10
NCH = BI * NSB
EP = NT * NCH * CH
ACH = 64
ABI = 32
NSB2 = 5
RPT = NP // NT
RB = 128

_mesh = plsc.VectorSubcoreMesh(core_axis_name="c", subcore_axis_name="s")


@functools.partial(
    pl.kernel,
    out_type=jax.ShapeDtypeStruct((4, NP, F), jnp.float32),
    mesh=_mesh,
    scratch_types=[
        pltpu.VMEM((BI, CH), jnp.int32),
        pltpu.VMEM((RB, F), jnp.float32),
        pltpu.VMEM_SHARED((NP, F), jnp.float32),
    ],
)
def _hist_kernel(spd, dpd, sdp, ddp, out, idx_v, val_v, hacc):
    c = lax.axis_index("c")
    s = lax.axis_index("s")
    row0 = s * RPT

    def fill(v):
        def body(r, _):
            for q in range(F // 16):
                val_v[r, pl.ds(q * 16, 16)] = jnp.full((16,), v, jnp.float32)
            return ()
        lax.fori_loop(0, RB, body, ())

    def one_hist(idx_hbm, o):
        fill(0.0)
        for j in range(RPT // RB):
            pltpu.sync_copy(val_v, hacc.at[pl.ds(row0 + j * RB, RB)])
        fill(1.0)
        plsc.subcore_barrier()

        def blk(b, _):
            pltpu.sync_copy(idx_hbm.at[s, b], idx_v)

            def body(k, _):
                pltpu.sync_copy(val_v, hacc.at[idx_v.at[k]], add=True)
                return ()
            lax.fori_loop(0, BI, body, ())
            return ()
        lax.fori_loop(0, NSB, blk, ())
        plsc.subcore_barrier()
        for j in range(RPT // RB):
            r = row0 + j * RB
            pltpu.sync_copy(hacc.at[pl.ds(r, RB)], out.at[o].at[pl.ds(r, RB)])
        plsc.subcore_barrier()

    @pl.when(c == 0)
    def _():
        one_hist(spd, 0)
        one_hist(dpd, 1)

    @pl.when(c == 1)
    def _():
        one_hist(sdp, 2)
        one_hist(ddp, 3)


@functools.partial(
    pl.kernel,
    out_type=jax.ShapeDtypeStruct((2, NP, F), jnp.float32),
    mesh=_mesh,
    scratch_types=[
        pltpu.VMEM((ABI, ACH), jnp.int32),
        pltpu.VMEM((ABI, ACH), jnp.int32),
        [pltpu.VMEM((ACH, F), jnp.float32)] * 4,
        pltpu.VMEM_SHARED((NP, F), jnp.float32),
        [pltpu.SemaphoreType.DMA] * 4,
        [pltpu.SemaphoreType.DMA] * 4,
    ],
)
def _agg1_kernel(table, src_hbm, dst_hbm, out,
                 src_v, dst_v, rows, acc, sg, ss):
    c = lax.axis_index("c")
    s = lax.axis_index("s")
    row0 = s * RPT

    def zbody(r, _):
        for q in range(F // 16):
            rows[0][r, pl.ds(q * 16, 16)] = jnp.zeros((16,), jnp.float32)
        return ()
    lax.fori_loop(0, ACH, zbody, ())
    for j in range(RPT // ACH):
        pltpu.sync_copy(rows[0], acc.at[pl.ds(row0 + j * ACH, ACH)])
    plsc.subcore_barrier()

    def gfire(k, b):
        pltpu.async_copy(table.at[src_v.at[k]], rows[b], sg[b])

    def gwait(k, b):
        pltpu.make_async_copy(table.at[src_v.at[k]], rows[b], sg[b]).wait()

    def blk(bb, _):
        pltpu.sync_copy(src_hbm.at[c, s, bb], src_v)
        pltpu.sync_copy(dst_hbm.at[c, s, bb], dst_v)
        for b in range(4):
            gfire(b, b)

        def quad(g, _):
            k = g * 4
            scs = []
            for b in range(4):
                gwait(k + b, b)
                scs.append(pltpu.async_copy(
                    rows[b], acc.at[dst_v.at[k + b]], ss[b], add=True))
            for b in range(4):
                scs[b].wait()
                gfire(k + 4 + b, b)
            return ()
        lax.fori_loop(0, ABI // 4 - 1, quad, ())

        kt = ABI - 4
        scs = []
        for b in range(4):
            gwait(kt + b, b)
            scs.append(pltpu.async_copy(
                rows[b], acc.at[dst_v.at[kt + b]], ss[b], add=True))
        for b in range(4):
            scs[b].wait()
        return ()
    lax.fori_loop(0, NSB2, blk, ())
    plsc.subcore_barrier()
    for j in range(RPT // RB):
        r = row0 + j * RB
        pltpu.sync_copy(acc.at[pl.ds(r, RB)], out.at[c].at[pl.ds(r, RB)])


_R = 2048


def _prep_body(xp_ref, xd_ref, h_ref, xps_ref, xds_ref,
               odp_ref, idd_ref, odd_ref, idp_ref):
    h = h_ref[...]
    sc = lax.rsqrt(jnp.maximum(h[:, :, 0:1], 1.0))
    odp, idd, odd, idp = sc[0], sc[1], sc[2], sc[3]
    xps_ref[...] = xp_ref[...] * odp
    xds_ref[...] = xd_ref[...] * odd
    odp_ref[...] = odp
    idd_ref[...] = idd
    odd_ref[...] = odd
    idp_ref[...] = idp


_prep = pl.pallas_call(
    _prep_body,
    grid=(NP // _R,),
    in_specs=[
        pl.BlockSpec((_R, F), lambda i: (i, 0)),
        pl.BlockSpec((_R, F), lambda i: (i, 0)),
        pl.BlockSpec((4, _R, 1), lambda i: (0, i, 0)),
    ],
    out_specs=[
        pl.BlockSpec((_R, F), lambda i: (i, 0)),
        pl.BlockSpec((_R, F), lambda i: (i, 0)),
        pl.BlockSpec((_R, 1), lambda i: (i, 0)),
        pl.BlockSpec((_R, 1), lambda i: (i, 0)),
        pl.BlockSpec((_R, 1), lambda i: (i, 0)),
        pl.BlockSpec((_R, 1), lambda i: (i, 0)),
    ],
    out_shape=[
        jax.ShapeDtypeStruct((NP, F), jnp.float32),
        jax.ShapeDtypeStruct((NP, F), jnp.float32),
        jax.ShapeDtypeStruct((NP, 1), jnp.float32),
        jax.ShapeDtypeStruct((NP, 1), jnp.float32),
        jax.ShapeDtypeStruct((NP, 1), jnp.float32),
        jax.ShapeDtypeStruct((NP, 1), jnp.float32),
    ],
)


def _half_body(prescale, a_ref, w_ref, b_ref, isc_ref, osc_ref, z_ref):
    a = a_ref[0] + a_ref[1]
    m = jnp.dot(a, w_ref[...], preferred_element_type=jnp.float32)
    y = jnp.maximum(m * isc_ref[...] + b_ref[...], 0.0)
    if prescale:
        y = y * osc_ref[...]
    z_ref[...] = y


def _make_half_layer(prescale):
    return pl.pallas_call(
        functools.partial(_half_body, prescale),
        grid=(NP // _R,),
        in_specs=[
            pl.BlockSpec((2, _R, F), lambda i: (0, i, 0)),
            pl.BlockSpec((F, F), lambda i: (0, 0)),
            pl.BlockSpec((1, F), lambda i: (0, 0)),
            pl.BlockSpec((_R, 1), lambda i: (i, 0)),
            pl.BlockSpec((_R, 1), lambda i: (i, 0)),
        ],
        out_specs=pl.BlockSpec((_R, F), lambda i: (i, 0)),
        out_shape=jax.ShapeDtypeStruct((NP, F), jnp.float32),
    )


_layer_pre = _make_half_layer(True)
_layer_post = _make_half_layer(False)


def _pad_edges(idx):
    return jnp.pad(idx, (0, EP - E), constant_values=N)


def kernel(xp, xd, edge_pd, edge_dp, W1_pd, b1_pd, W1_dp, b1_dp,
           W2_pd, b2_pd, W2_dp, b2_dp, W3_pd, b3_pd, W3_dp, b3_dp):
    flat = [_pad_edges(edge_pd[0]), _pad_edges(edge_pd[1]),
            _pad_edges(edge_dp[0]), _pad_edges(edge_dp[1])]
    hshape = (NT, NSB, BI, CH)
    ashape = (2, NT, NSB2, ABI, ACH)
    spd_h, dpd_h, sdp_h, ddp_h = (a.reshape(hshape) for a in flat)
    spd, dpd, sdp, ddp = (a.reshape(ashape) for a in flat)
    xp_p = jnp.pad(xp, ((0, NP - N), (0, 0)))
    xd_p = jnp.pad(xd, ((0, NP - N), (0, 0)))

    hist = _hist_kernel(spd_h, dpd_h, sdp_h, ddp_h)[:, :, :1]
    xps, xds, odp_s, idd_s, odd_s, idp_s = _prep(xp_p, xd_p, hist)

    b1 = (b1_pd.reshape(1, F), b1_dp.reshape(1, F))
    b2 = (b2_pd.reshape(1, F), b2_dp.reshape(1, F))
    b3 = (b3_pd.reshape(1, F), b3_dp.reshape(1, F))

    a1 = _agg1_kernel(xps, spd, dpd)
    bb1 = _agg1_kernel(xds, sdp, ddp)
    z1d = _layer_pre(a1, W1_pd, b1[0], idd_s, odd_s)
    bb2 = _agg1_kernel(z1d, sdp, ddp)
    z1p = _layer_pre(bb1, W1_dp, b1[1], idp_s, odp_s)
    a2 = _agg1_kernel(z1p, spd, dpd)
    z2p = _layer_pre(bb2, W2_dp, b2[1], idp_s, odp_s)
    a3 = _agg1_kernel(z2p, spd, dpd)
    z2d = _layer_pre(a2, W2_pd, b2[0], idd_s, odd_s)
    bb3 = _agg1_kernel(z2d, sdp, ddp)
    h3d = _layer_post(a3, W3_pd, b3[0], idd_s, idd_s)
    h3p = _layer_post(bb3, W3_dp, b3[1], idp_s, idp_s)
    return (h3p[:N], h3d[:N])


# speedup vs baseline: 1.4352x; 1.4352x over previous
import functools

import jax
import jax.numpy as jnp
from jax import lax
from jax.experimental import pallas as pl
from jax.experimental.pallas import tpu as pltpu
from jax.experimental.pallas import tpu_sc as plsc

N = 10000
NP = 10240
E = 320000
F = 128
NT = 16
CH = 128
BI = 16
NSB = 10
NCH = BI * NSB
EP = NT * NCH * CH
ACH = 64
ABI = 32
RPT = NP // NT
RB = 128
HW = 16

_mesh = plsc.VectorSubcoreMesh(core_axis_name="c", subcore_axis_name="s")


@functools.partial(
    pl.kernel,
    out_type=jax.ShapeDtypeStruct((4, NP, F), jnp.float32),
    mesh=_mesh,
    scratch_types=[
        pltpu.VMEM((BI, CH), jnp.int32),
        pltpu.VMEM((RB, F), jnp.float32),
        pltpu.VMEM_SHARED((NP, F), jnp.float32),
    ],
)
def _hist_kernel(spd, dpd, sdp, ddp, out, idx_v, val_v, hacc):
    c = lax.axis_index("c")
    s = lax.axis_index("s")
    row0 = s * RPT

    def fill(v):
        def body(r, _):
            for q in range(F // 16):
                val_v[r, pl.ds(q * 16, 16)] = jnp.full((16,), v, jnp.float32)
            return ()
        lax.fori_loop(0, RB, body, ())

    def one_hist(idx_hbm, o):
        fill(0.0)
        for j in range(RPT // RB):
            pltpu.sync_copy(val_v, hacc.at[pl.ds(row0 + j * RB, RB)])
        fill(1.0)
        plsc.subcore_barrier()

        def blk(b, _):
            pltpu.sync_copy(idx_hbm.at[s, b], idx_v)

            def body(k, _):
                pltpu.sync_copy(val_v, hacc.at[idx_v.at[k]], add=True)
                return ()
            lax.fori_loop(0, BI, body, ())
            return ()
        lax.fori_loop(0, NSB, blk, ())
        plsc.subcore_barrier()
        for j in range(RPT // RB):
            r = row0 + j * RB
            pltpu.sync_copy(hacc.at[pl.ds(r, RB)], out.at[o].at[pl.ds(r, RB)])
        plsc.subcore_barrier()

    @pl.when(c == 0)
    def _():
        one_hist(spd, 0)
        one_hist(dpd, 1)

    @pl.when(c == 1)
    def _():
        one_hist(sdp, 2)
        one_hist(ddp, 3)


@functools.partial(
    pl.kernel,
    out_type=(
        jax.ShapeDtypeStruct((NP, F), jnp.float32),
        jax.ShapeDtypeStruct((NP, F), jnp.float32),
    ),
    mesh=_mesh,
    scratch_types=[
        pltpu.VMEM((ABI, ACH), jnp.int32),
        pltpu.VMEM((ABI, ACH), jnp.int32),
        [pltpu.VMEM((ACH, F), jnp.float32)] * 4,
        pltpu.VMEM_SHARED((NP, F), jnp.float32),
        [pltpu.SemaphoreType.DMA] * 4,
        [pltpu.SemaphoreType.DMA] * 4,
    ],
)
def _agg_kernel(xa, xb, spd, dpd, sdp, ddp, out_d, out_p,
                src_v, dst_v, rows, acc, sg, ss):
    c = lax.axis_index("c")
    s = lax.axis_index("s")
    row0 = s * RPT

    def run(table, src_hbm, dst_hbm, out):
        def zbody(r, _):
            for q in range(F // 16):
                rows[0][r, pl.ds(q * 16, 16)] = jnp.zeros((16,), jnp.float32)
            return ()
        lax.fori_loop(0, ACH, zbody, ())
        for j in range(RPT // ACH):
            pltpu.sync_copy(rows[0], acc.at[pl.ds(row0 + j * ACH, ACH)])
        plsc.subcore_barrier()

        def gfire(k, b):
            pltpu.async_copy(table.at[src_v.at[k]], rows[b], sg[b])

        def gwait(k, b):
            pltpu.make_async_copy(table.at[src_v.at[k]], rows[b], sg[b]).wait()

        def blk(bb, _):
            pltpu.sync_copy(src_hbm.at[s, bb], src_v)
            pltpu.sync_copy(dst_hbm.at[s, bb], dst_v)
            for b in range(4):
                gfire(b, b)

            def quad(g, _):
                k = g * 4
                scs = []
                for b in range(4):
                    gwait(k + b, b)
                    scs.append(pltpu.async_copy(
                        rows[b], acc.at[dst_v.at[k + b]], ss[b], add=True))
                for b in range(4):
                    scs[b].wait()
                    gfire(k + 4 + b, b)
                return ()
            lax.fori_loop(0, ABI // 4 - 1, quad, ())

            kt = ABI - 4
            scs = []
            for b in range(4):
                gwait(kt + b, b)
                scs.append(pltpu.async_copy(
                    rows[b], acc.at[dst_v.at[kt + b]], ss[b], add=True))
            for b in range(4):
                scs[b].wait()
            return ()
        lax.fori_loop(0, NSB, blk, ())
        plsc.subcore_barrier()
        for j in range(RPT // RB):
            r = row0 + j * RB
            pltpu.sync_copy(acc.at[pl.ds(r, RB)], out.at[pl.ds(r, RB)])

    @pl.when(c == 0)
    def _():
        run(xa, spd, dpd, out_d)

    @pl.when(c == 1)
    def _():
        run(xb, sdp, ddp, out_p)


_R = 2048


def _prep_body(xp_ref, xd_ref, h_ref, xps_ref, xds_ref,
               odp_ref, idd_ref, odd_ref, idp_ref):
    h = h_ref[...]
    sc = lax.rsqrt(jnp.maximum(h[:, :, 0:1], 1.0))
    odp, idd, odd, idp = sc[0], sc[1], sc[2], sc[3]
    xps_ref[...] = xp_ref[...] * odp
    xds_ref[...] = xd_ref[...] * odd
    odp_ref[...] = odp
    idd_ref[...] = idd
    odd_ref[...] = odd
    idp_ref[...] = idp


_prep = pl.pallas_call(
    _prep_body,
    grid=(NP // _R,),
    in_specs=[
        pl.BlockSpec((_R, F), lambda i: (i, 0)),
        pl.BlockSpec((_R, F), lambda i: (i, 0)),
        pl.BlockSpec((4, _R, 1), lambda i: (0, i, 0)),
    ],
    out_specs=[
        pl.BlockSpec((_R, F), lambda i: (i, 0)),
        pl.BlockSpec((_R, F), lambda i: (i, 0)),
        pl.BlockSpec((_R, 1), lambda i: (i, 0)),
        pl.BlockSpec((_R, 1), lambda i: (i, 0)),
        pl.BlockSpec((_R, 1), lambda i: (i, 0)),
        pl.BlockSpec((_R, 1), lambda i: (i, 0)),
    ],
    out_shape=[
        jax.ShapeDtypeStruct((NP, F), jnp.float32),
        jax.ShapeDtypeStruct((NP, F), jnp.float32),
        jax.ShapeDtypeStruct((NP, 1), jnp.float32),
        jax.ShapeDtypeStruct((NP, 1), jnp.float32),
        jax.ShapeDtypeStruct((NP, 1), jnp.float32),
        jax.ShapeDtypeStruct((NP, 1), jnp.float32),
    ],
)


def _layer_body(prescale, ad_ref, ap_ref, wd_ref, wp_ref, bd_ref, bp_ref,
                idd_ref, idp_ref, odd_ref, odp_ref, zd_ref, zp_ref):
    md = jnp.dot(ad_ref[...], wd_ref[...], preferred_element_type=jnp.float32)
    yd = jnp.maximum(md * idd_ref[...] + bd_ref[...], 0.0)
    if prescale:
        yd = yd * odd_ref[...]
    zd_ref[...] = yd
    mp = jnp.dot(ap_ref[...], wp_ref[...], preferred_element_type=jnp.float32)
    yp = jnp.maximum(mp * idp_ref[...] + bp_ref[...], 0.0)
    if prescale:
        yp = yp * odp_ref[...]
    zp_ref[...] = yp


def _make_layer(prescale):
    return pl.pallas_call(
        functools.partial(_layer_body, prescale),
        grid=(NP // _R,),
        in_specs=[
            pl.BlockSpec((_R, F), lambda i: (i, 0)),
            pl.BlockSpec((_R, F), lambda i: (i, 0)),
            pl.BlockSpec((F, F), lambda i: (0, 0)),
            pl.BlockSpec((F, F), lambda i: (0, 0)),
            pl.BlockSpec((1, F), lambda i: (0, 0)),
            pl.BlockSpec((1, F), lambda i: (0, 0)),
            pl.BlockSpec((_R, 1), lambda i: (i, 0)),
            pl.BlockSpec((_R, 1), lambda i: (i, 0)),
            pl.BlockSpec((_R, 1), lambda i: (i, 0)),
            pl.BlockSpec((_R, 1), lambda i: (i, 0)),
        ],
        out_specs=[
            pl.BlockSpec((_R, F), lambda i: (i, 0)),
            pl.BlockSpec((_R, F), lambda i: (i, 0)),
        ],
        out_shape=[
            jax.ShapeDtypeStruct((NP, F), jnp.float32),
            jax.ShapeDtypeStruct((NP, F), jnp.float32),
        ],
    )


_layer_pre = _make_layer(True)
_layer_post = _make_layer(False)


def _pad_edges(idx):
    return jnp.pad(idx, (0, EP - E), constant_values=N)


def kernel(xp, xd, edge_pd, edge_dp, W1_pd, b1_pd, W1_dp, b1_dp,
           W2_pd, b2_pd, W2_dp, b2_dp, W3_pd, b3_pd, W3_dp, b3_dp):
    flat = [_pad_edges(edge_pd[0]), _pad_edges(edge_pd[1]),
            _pad_edges(edge_dp[0]), _pad_edges(edge_dp[1])]
    hshape = (NT, NSB, BI, CH)
    ashape = (NT, NSB, ABI, ACH)
    spd_h, dpd_h, sdp_h, ddp_h = (a.reshape(hshape) for a in flat)
    spd, dpd, sdp, ddp = (a.reshape(ashape) for a in flat)
    xp_p = jnp.pad(xp, ((0, NP - N), (0, 0)))
    xd_p = jnp.pad(xd, ((0, NP - N), (0, 0)))

    hist = _hist_kernel(spd_h, dpd_h, sdp_h, ddp_h)[:, :, :1]
    xps, xds, odp_s, idd_s, odd_s, idp_s = _prep(xp_p, xd_p, hist)

    a1d, a1p = _agg_kernel(xps, xds, spd, dpd, sdp, ddp)
    z1d, z1p = _layer_pre(a1d, a1p, W1_pd, W1_dp,
                          b1_pd.reshape(1, F), b1_dp.reshape(1, F),
                          idd_s, idp_s, odd_s, odp_s)
    a2d, a2p = _agg_kernel(z1p, z1d, spd, dpd, sdp, ddp)
    z2d, z2p = _layer_pre(a2d, a2p, W2_pd, W2_dp,
                          b2_pd.reshape(1, F), b2_dp.reshape(1, F),
                          idd_s, idp_s, odd_s, odp_s)
    a3d, a3p = _agg_kernel(z2p, z2d, spd, dpd, sdp, ddp)
    h3d, h3p = _layer_post(a3d, a3p, W3_pd, W3_dp,
                           b3_pd.reshape(1, F), b3_dp.reshape(1, F),
                           idd_s, idp_s, odd_s, odp_s)
    return (h3p[:N], h3d[:N])


# speedup vs baseline: 1.4531x; 1.0125x over previous
import functools

import jax
import jax.numpy as jnp
from jax import lax
from jax.experimental import pallas as pl
from jax.experimental.pallas import tpu as pltpu
from jax.experimental.pallas import tpu_sc as plsc

N = 10000
NP = 10240
E = 320000
F = 128
NT = 16
CH = 128
BI = 16
NSB = 10
NCH = BI * NSB
EP = NT * NCH * CH
ACH = 64
ABI = 64
ANSB = 5
RPT = NP // NT
RB = 128
HW = 16

_mesh = plsc.VectorSubcoreMesh(core_axis_name="c", subcore_axis_name="s")


@functools.partial(
    pl.kernel,
    out_type=jax.ShapeDtypeStruct((4, NP, F), jnp.float32),
    mesh=_mesh,
    scratch_types=[
        pltpu.VMEM((BI, CH), jnp.int32),
        pltpu.VMEM((RB, F), jnp.float32),
        pltpu.VMEM_SHARED((NP, F), jnp.float32),
    ],
)
def _hist_kernel(spd, dpd, sdp, ddp, out, idx_v, val_v, hacc):
    c = lax.axis_index("c")
    s = lax.axis_index("s")
    row0 = s * RPT

    def fill(v):
        def body(r, _):
            for q in range(F // 16):
                val_v[r, pl.ds(q * 16, 16)] = jnp.full((16,), v, jnp.float32)
            return ()
        lax.fori_loop(0, RB, body, ())

    def one_hist(idx_hbm, o):
        fill(0.0)
        for j in range(RPT // RB):
            pltpu.sync_copy(val_v, hacc.at[pl.ds(row0 + j * RB, RB)])
        fill(1.0)
        plsc.subcore_barrier()

        def blk(b, _):
            pltpu.sync_copy(idx_hbm.at[s, b], idx_v)

            def body(k, _):
                pltpu.sync_copy(val_v, hacc.at[idx_v.at[k]], add=True)
                return ()
            lax.fori_loop(0, BI, body, ())
            return ()
        lax.fori_loop(0, NSB, blk, ())
        plsc.subcore_barrier()
        for j in range(RPT // RB):
            r = row0 + j * RB
            pltpu.sync_copy(hacc.at[pl.ds(r, RB)], out.at[o].at[pl.ds(r, RB)])
        plsc.subcore_barrier()

    @pl.when(c == 0)
    def _():
        one_hist(spd, 0)
        one_hist(dpd, 1)

    @pl.when(c == 1)
    def _():
        one_hist(sdp, 2)
        one_hist(ddp, 3)


@functools.partial(
    pl.kernel,
    out_type=(
        jax.ShapeDtypeStruct((NP, F), jnp.float32),
        jax.ShapeDtypeStruct((NP, F), jnp.float32),
    ),
    mesh=_mesh,
    scratch_types=[
        pltpu.VMEM((ABI, ACH), jnp.int32),
        pltpu.VMEM((ABI, ACH), jnp.int32),
        [pltpu.VMEM((ACH, F), jnp.float32)] * 4,
        pltpu.VMEM_SHARED((NP, F), jnp.float32),
        [pltpu.SemaphoreType.DMA] * 4,
        [pltpu.SemaphoreType.DMA] * 4,
    ],
)
def _agg_kernel(xa, xb, spd, dpd, sdp, ddp, out_d, out_p,
                src_v, dst_v, rows, acc, sg, ss):
    c = lax.axis_index("c")
    s = lax.axis_index("s")
    row0 = s * RPT

    def run(table, src_hbm, dst_hbm, out):
        def zbody(r, _):
            for q in range(F // 16):
                rows[0][r, pl.ds(q * 16, 16)] = jnp.zeros((16,), jnp.float32)
            return ()
        lax.fori_loop(0, ACH, zbody, ())
        for j in range(RPT // ACH):
            pltpu.sync_copy(rows[0], acc.at[pl.ds(row0 + j * ACH, ACH)])
        plsc.subcore_barrier()

        def gfire(k, b):
            pltpu.async_copy(table.at[src_v.at[k]], rows[b], sg[b])

        def gwait(k, b):
            pltpu.make_async_copy(table.at[src_v.at[k]], rows[b], sg[b]).wait()

        def blk(bb, _):
            pltpu.sync_copy(src_hbm.at[s, bb], src_v)
            pltpu.sync_copy(dst_hbm.at[s, bb], dst_v)
            for b in range(4):
                gfire(b, b)

            def quad(g, _):
                k = g * 4
                scs = []
                for b in range(4):
                    gwait(k + b, b)
                    scs.append(pltpu.async_copy(
                        rows[b], acc.at[dst_v.at[k + b]], ss[b], add=True))
                for b in range(4):
                    scs[b].wait()
                    gfire(k + 4 + b, b)
                return ()
            lax.fori_loop(0, ABI // 4 - 1, quad, ())

            kt = ABI - 4
            scs = []
            for b in range(4):
                gwait(kt + b, b)
                scs.append(pltpu.async_copy(
                    rows[b], acc.at[dst_v.at[kt + b]], ss[b], add=True))
            for b in range(4):
                scs[b].wait()
            return ()
        lax.fori_loop(0, ANSB, blk, ())
        plsc.subcore_barrier()
        for j in range(RPT // RB):
            r = row0 + j * RB
            pltpu.sync_copy(acc.at[pl.ds(r, RB)], out.at[pl.ds(r, RB)])

    @pl.when(c == 0)
    def _():
        run(xa, spd, dpd, out_d)

    @pl.when(c == 1)
    def _():
        run(xb, sdp, ddp, out_p)


_R = 2048


def _prep_body(xp_ref, xd_ref, h_ref, xps_ref, xds_ref,
               odp_ref, idd_ref, odd_ref, idp_ref):
    h = h_ref[...]
    sc = lax.rsqrt(jnp.maximum(h[:, :, 0:1], 1.0))
    odp, idd, odd, idp = sc[0], sc[1], sc[2], sc[3]
    xps_ref[...] = xp_ref[...] * odp
    xds_ref[...] = xd_ref[...] * odd
    odp_ref[...] = odp
    idd_ref[...] = idd
    odd_ref[...] = odd
    idp_ref[...] = idp


_prep = pl.pallas_call(
    _prep_body,
    grid=(NP // _R,),
    in_specs=[
        pl.BlockSpec((_R, F), lambda i: (i, 0)),
        pl.BlockSpec((_R, F), lambda i: (i, 0)),
        pl.BlockSpec((4, _R, 1), lambda i: (0, i, 0)),
    ],
    out_specs=[
        pl.BlockSpec((_R, F), lambda i: (i, 0)),
        pl.BlockSpec((_R, F), lambda i: (i, 0)),
        pl.BlockSpec((_R, 1), lambda i: (i, 0)),
        pl.BlockSpec((_R, 1), lambda i: (i, 0)),
        pl.BlockSpec((_R, 1), lambda i: (i, 0)),
        pl.BlockSpec((_R, 1), lambda i: (i, 0)),
    ],
    out_shape=[
        jax.ShapeDtypeStruct((NP, F), jnp.float32),
        jax.ShapeDtypeStruct((NP, F), jnp.float32),
        jax.ShapeDtypeStruct((NP, 1), jnp.float32),
        jax.ShapeDtypeStruct((NP, 1), jnp.float32),
        jax.ShapeDtypeStruct((NP, 1), jnp.float32),
        jax.ShapeDtypeStruct((NP, 1), jnp.float32),
    ],
)


def _layer_body(prescale, ad_ref, ap_ref, wd_ref, wp_ref, bd_ref, bp_ref,
                idd_ref, idp_ref, odd_ref, odp_ref, zd_ref, zp_ref):
    md = jnp.dot(ad_ref[...], wd_ref[...], preferred_element_type=jnp.float32)
    yd = jnp.maximum(md * idd_ref[...] + bd_ref[...], 0.0)
    if prescale:
        yd = yd * odd_ref[...]
    zd_ref[...] = yd
    mp = jnp.dot(ap_ref[...], wp_ref[...], preferred_element_type=jnp.float32)
    yp = jnp.maximum(mp * idp_ref[...] + bp_ref[...], 0.0)
    if prescale:
        yp = yp * odp_ref[...]
    zp_ref[...] = yp


def _make_layer(prescale):
    return pl.pallas_call(
        functools.partial(_layer_body, prescale),
        grid=(NP // _R,),
        in_specs=[
            pl.BlockSpec((_R, F), lambda i: (i, 0)),
            pl.BlockSpec((_R, F), lambda i: (i, 0)),
            pl.BlockSpec((F, F), lambda i: (0, 0)),
            pl.BlockSpec((F, F), lambda i: (0, 0)),
            pl.BlockSpec((1, F), lambda i: (0, 0)),
            pl.BlockSpec((1, F), lambda i: (0, 0)),
            pl.BlockSpec((_R, 1), lambda i: (i, 0)),
            pl.BlockSpec((_R, 1), lambda i: (i, 0)),
            pl.BlockSpec((_R, 1), lambda i: (i, 0)),
            pl.BlockSpec((_R, 1), lambda i: (i, 0)),
        ],
        out_specs=[
            pl.BlockSpec((_R, F), lambda i: (i, 0)),
            pl.BlockSpec((_R, F), lambda i: (i, 0)),
        ],
        out_shape=[
            jax.ShapeDtypeStruct((NP, F), jnp.float32),
            jax.ShapeDtypeStruct((NP, F), jnp.float32),
        ],
    )


_layer_pre = _make_layer(True)
_layer_post = _make_layer(False)


def _pad_edges(idx):
    return jnp.pad(idx, (0, EP - E), constant_values=N)


def kernel(xp, xd, edge_pd, edge_dp, W1_pd, b1_pd, W1_dp, b1_dp,
           W2_pd, b2_pd, W2_dp, b2_dp, W3_pd, b3_pd, W3_dp, b3_dp):
    flat = [_pad_edges(edge_pd[0]), _pad_edges(edge_pd[1]),
            _pad_edges(edge_dp[0]), _pad_edges(edge_dp[1])]
    hshape = (NT, NSB, BI, CH)
    ashape = (NT, ANSB, ABI, ACH)
    spd_h, dpd_h, sdp_h, ddp_h = (a.reshape(hshape) for a in flat)
    spd, dpd, sdp, ddp = (a.reshape(ashape) for a in flat)
    xp_p = jnp.pad(xp, ((0, NP - N), (0, 0)))
    xd_p = jnp.pad(xd, ((0, NP - N), (0, 0)))

    hist = _hist_kernel(spd_h, dpd_h, sdp_h, ddp_h)[:, :, :1]
    xps, xds, odp_s, idd_s, odd_s, idp_s = _prep(xp_p, xd_p, hist)

    a1d, a1p = _agg_kernel(xps, xds, spd, dpd, sdp, ddp)
    z1d, z1p = _layer_pre(a1d, a1p, W1_pd, W1_dp,
                          b1_pd.reshape(1, F), b1_dp.reshape(1, F),
                          idd_s, idp_s, odd_s, odp_s)
    a2d, a2p = _agg_kernel(z1p, z1d, spd, dpd, sdp, ddp)
    z2d, z2p = _layer_pre(a2d, a2p, W2_pd, W2_dp,
                          b2_pd.reshape(1, F), b2_dp.reshape(1, F),
                          idd_s, idp_s, odd_s, odp_s)
    a3d, a3p = _agg_kernel(z2p, z2d, spd, dpd, sdp, ddp)
    h3d, h3p = _layer_post(a3d, a3p, W3_pd, W3_dp,
                           b3_pd.reshape(1, F), b3_dp.reshape(1, F),
                           idd_s, idp_s, odd_s, odp_s)
    return (h3p[:N], h3d[:N])
